# baseline (speedup 1.0000x reference)
import functools

import jax
import jax.numpy as jnp
from jax import lax
from jax.experimental import pallas as pl
from jax.experimental.pallas import tpu as pltpu
from jax.experimental.pallas import tpu_sc as plsc

B = 16384
F = 128
NC = 2
NS = 16
NW = NC * NS
BPW = B // NW
C = 128
NCHUNK = BPW // C
NBUF = 2
L = 16
MPAD = L + 1


def _dot_body(uid_hbm, iid_hbm, p_hbm, q_hbm, out_hbm,
              idxu_v, idxi_v, pbufs, qbufs, mat, out_v, sems):
    wid = lax.axis_index("s") * NC + lax.axis_index("c")
    base = wid * BPW
    pltpu.sync_copy(uid_hbm.at[pl.ds(base, BPW)], idxu_v)
    pltpu.sync_copy(iid_hbm.at[pl.ds(base, BPW)], idxi_v)
    lanes = lax.iota(jnp.int32, L)

    def start(c):
        s = c % NBUF
        sl = pl.ds(c * C, C)
        return (
            pltpu.async_copy(p_hbm.at[idxu_v.at[sl]], pbufs.at[s], sems.at[s]),
            pltpu.async_copy(q_hbm.at[idxi_v.at[sl]], qbufs.at[s], sems.at[s]),
        )

    inflight = start(0)
    for c in range(NCHUNK):
        s = c % NBUF
        for cp in inflight:
            cp.wait()
        if c + 1 < NCHUNK:
            inflight = start(c + 1)
        pbuf, qbuf = pbufs.at[s], qbufs.at[s]
        c0 = c * C

        def group(g, _):
            r0 = g * L
            for rr in range(L):
                r = r0 + rr
                t = [pbuf[r, pl.ds(j * L, L)] * qbuf[r, pl.ds(j * L, L)]
                     for j in range(F // L)]
                t = [t[0] + t[1], t[2] + t[3], t[4] + t[5], t[6] + t[7]]
                t = [t[0] + t[1], t[2] + t[3]]
                mat[rr, pl.ds(0, L)] = t[0] + t[1]
            cols = [plsc.load_gather(mat, [lanes, jnp.full((L,), l, jnp.int32)])
                    for l in range(L)]
            for step in (8, 4, 2, 1):
                cols = [cols[i] + cols[i + step] for i in range(step)]
            out_v[pl.ds(c0 + r0, L)] = cols[0]
            return 0

        lax.fori_loop(0, C // L, group, 0)
    pltpu.sync_copy(out_v, out_hbm.at[pl.ds(base, BPW)])


W = 16
NU = 1_000_000 // NW
NI = 100_000 // NW


def _repack_body(bu_hbm, bi_hbm, bu_o, bi_o, stage_a, stage_b, sem):
    wid = lax.axis_index("s") * NC + lax.axis_index("c")

    def pump(src, dst, base, nchunk):
        stages = (stage_a, stage_b)
        cp = pltpu.async_copy(src.at[pl.ds(base, NI)], stages[0], sem)
        for k in range(nchunk):
            cp.wait()
            if k + 1 < nchunk:
                nxt = pltpu.async_copy(
                    src.at[pl.ds(base + (k + 1) * NI, NI)],
                    stages[(k + 1) % 2], sem)
            pltpu.sync_copy(stages[k % 2], dst.at[pl.ds(base + k * NI, NI)])
            if k + 1 < nchunk:
                cp = nxt

    pump(bu_hbm, bu_o, wid * NU, NU // NI)
    pump(bi_hbm, bi_o, wid * NI, 1)


def _bias_body(uid_hbm, iid_hbm, bu_hbm, bi_hbm, dots_hbm, out_hbm,
               idxu_v, idxi_v, tidxu, tidxi, colu_v, coli_v,
               bu_rows, bi_rows, dots_v, sem):
    wid = lax.axis_index("s") * NC + lax.axis_index("c")
    base = wid * BPW
    pltpu.sync_copy(uid_hbm.at[pl.ds(base, BPW)], idxu_v)
    pltpu.sync_copy(iid_hbm.at[pl.ds(base, BPW)], idxi_v)

    for j in range(BPW // L):
        sl = pl.ds(j * L, L)
        u = idxu_v[sl]
        i = idxi_v[sl]
        tidxu[j // (C // L), pl.ds((j % (C // L)) * L, L)] = u >> 4
        tidxi[j // (C // L), pl.ds((j % (C // L)) * L, L)] = i >> 4
        colu_v[sl] = u & (W - 1)
        coli_v[sl] = i & (W - 1)

    cps = []
    for k in range(BPW // C):
        sl = pl.ds(k * C, C)
        cps.append(pltpu.async_copy(bu_hbm.at[tidxu.at[k]],
                                    bu_rows.at[pl.ds(k * C, C)], sem))
        cps.append(pltpu.async_copy(bi_hbm.at[tidxi.at[k]],
                                    bi_rows.at[pl.ds(k * C, C)], sem))
    pltpu.sync_copy(dots_hbm.at[pl.ds(base, BPW)], dots_v)
    for cp in cps:
        cp.wait()
    lanes = lax.iota(jnp.int32, L)

    def group(g, _):
        eidx = g * L + lanes
        res = (dots_v[pl.ds(g * L, L)]
               + plsc.load_gather(bu_rows, [eidx, colu_v[pl.ds(g * L, L)]])
               + plsc.load_gather(bi_rows, [eidx, coli_v[pl.ds(g * L, L)]]))
        dots_v[pl.ds(g * L, L)] = res
        return 0

    lax.fori_loop(0, BPW // L, group, 0)
    pltpu.sync_copy(dots_v, out_hbm.at[pl.ds(base, BPW)])


def kernel(user_id, item_id, P, Q, user_bias, item_bias):
    mesh = plsc.VectorSubcoreMesh(core_axis_name="c", subcore_axis_name="s",
                                  num_cores=NC, num_subcores=NS)
    dot_run = functools.partial(
        pl.kernel,
        out_type=jax.ShapeDtypeStruct((B,), jnp.float32),
        mesh=mesh,
        compiler_params=pltpu.CompilerParams(needs_layout_passes=False),
        scratch_types=[
            pltpu.VMEM((BPW,), jnp.int32),
            pltpu.VMEM((BPW,), jnp.int32),
            pltpu.VMEM((NBUF, C, F), jnp.float32),
            pltpu.VMEM((NBUF, C, F), jnp.float32),
            pltpu.VMEM((L, MPAD), jnp.float32),
            pltpu.VMEM((BPW,), jnp.float32),
            pltpu.SemaphoreType.DMA((NBUF,)),
        ],
    )(_dot_body)
    bias_run = functools.partial(
        pl.kernel,
        out_type=jax.ShapeDtypeStruct((B,), jnp.float32),
        mesh=mesh,
        compiler_params=pltpu.CompilerParams(needs_layout_passes=False,
                                             use_tc_tiling_on_sc=False),
        scratch_types=[
            pltpu.VMEM((BPW,), jnp.int32),
            pltpu.VMEM((BPW,), jnp.int32),
            pltpu.VMEM((BPW // C, C), jnp.int32),
            pltpu.VMEM((BPW // C, C), jnp.int32),
            pltpu.VMEM((BPW,), jnp.int32),
            pltpu.VMEM((BPW,), jnp.int32),
            pltpu.VMEM((BPW, W), jnp.float32),
            pltpu.VMEM((BPW, W), jnp.float32),
            pltpu.VMEM((BPW,), jnp.float32),
            pltpu.SemaphoreType.DMA,
        ],
    )(_bias_body)
    repack_run = functools.partial(
        pl.kernel,
        out_type=(jax.ShapeDtypeStruct((1_000_000, 1), jnp.float32),
                  jax.ShapeDtypeStruct((100_000, 1), jnp.float32)),
        mesh=mesh,
        compiler_params=pltpu.CompilerParams(needs_layout_passes=False,
                                             use_tc_tiling_on_sc=False),
        scratch_types=[
            pltpu.VMEM((NI, 1), jnp.float32),
            pltpu.VMEM((NI, 1), jnp.float32),
            pltpu.SemaphoreType.DMA,
        ],
    )(_repack_body)
    bu_c, bi_c = repack_run(user_bias, item_bias)
    dots = dot_run(user_id, item_id, P, Q)
    out = bias_run(user_id, item_id,
                   bu_c.reshape(-1, W), bi_c.reshape(-1, W), dots)
    return out.reshape(B, 1)


# speedup vs baseline: 23.3870x; 23.3870x over previous
import functools

import jax
import jax.numpy as jnp
from jax import lax
from jax.experimental import pallas as pl
from jax.experimental.pallas import tpu as pltpu
from jax.experimental.pallas import tpu_sc as plsc

B = 16384
F = 128
NC = 2
NS = 16
NW = NC * NS
BPW = B // NW
C = 128
NCHUNK = BPW // C
NBUF = 2
L = 16
MPAD = L + 1


def _dot_body(uid_hbm, iid_hbm, p_hbm, q_hbm, out_hbm,
              idxu_v, idxi_v, pbufs, qbufs, mat, out_v, sems):
    wid = lax.axis_index("s") * NC + lax.axis_index("c")
    base = wid * BPW
    pltpu.sync_copy(uid_hbm.at[pl.ds(base, BPW)], idxu_v)
    pltpu.sync_copy(iid_hbm.at[pl.ds(base, BPW)], idxi_v)
    lanes = lax.iota(jnp.int32, L)

    def start(c):
        s = c % NBUF
        sl = pl.ds(c * C, C)
        return (
            pltpu.async_copy(p_hbm.at[idxu_v.at[sl]], pbufs.at[s], sems.at[s]),
            pltpu.async_copy(q_hbm.at[idxi_v.at[sl]], qbufs.at[s], sems.at[s]),
        )

    inflight = start(0)
    for c in range(NCHUNK):
        s = c % NBUF
        for cp in inflight:
            cp.wait()
        if c + 1 < NCHUNK:
            inflight = start(c + 1)
        pbuf, qbuf = pbufs.at[s], qbufs.at[s]
        c0 = c * C

        def group(g, _):
            r0 = g * L
            for rr in range(L):
                r = r0 + rr
                t = [pbuf[r, pl.ds(j * L, L)] * qbuf[r, pl.ds(j * L, L)]
                     for j in range(F // L)]
                t = [t[0] + t[1], t[2] + t[3], t[4] + t[5], t[6] + t[7]]
                t = [t[0] + t[1], t[2] + t[3]]
                mat[rr, pl.ds(0, L)] = t[0] + t[1]
            cols = [plsc.load_gather(mat, [lanes, jnp.full((L,), l, jnp.int32)])
                    for l in range(L)]
            for step in (8, 4, 2, 1):
                cols = [cols[i] + cols[i + step] for i in range(step)]
            out_v[pl.ds(c0 + r0, L)] = cols[0]
            return 0

        lax.fori_loop(0, C // L, group, 0)
    pltpu.sync_copy(out_v, out_hbm.at[pl.ds(base, BPW)])


def _bias_body(uid_hbm, iid_hbm, bu_hbm, bi_hbm, dots_hbm, out_hbm,
               idxu_v, idxi_v, bu_v, bi_v, dots_v, sem):
    wid = lax.axis_index("s") * NC + lax.axis_index("c")
    base = wid * BPW
    pltpu.sync_copy(uid_hbm.at[pl.ds(base, BPW)], idxu_v)
    pltpu.sync_copy(iid_hbm.at[pl.ds(base, BPW)], idxi_v)

    cps = []
    for k in range(BPW // C):
        sl = pl.ds(k * C, C)
        cps.append(pltpu.async_copy(bu_hbm.at[idxu_v.at[sl]], bu_v.at[sl], sem))
        cps.append(pltpu.async_copy(bi_hbm.at[idxi_v.at[sl]], bi_v.at[sl], sem))
    pltpu.sync_copy(dots_hbm.at[pl.ds(base, BPW)], dots_v)
    for cp in cps:
        cp.wait()

    def group(g, _):
        res = (dots_v[pl.ds(g * L, L)]
               + bu_v[pl.ds(g * L, L)] + bi_v[pl.ds(g * L, L)])
        dots_v[pl.ds(g * L, L)] = res
        return 0

    lax.fori_loop(0, BPW // L, group, 0)
    pltpu.sync_copy(dots_v, out_hbm.at[pl.ds(base, BPW)])


def kernel(user_id, item_id, P, Q, user_bias, item_bias):
    mesh = plsc.VectorSubcoreMesh(core_axis_name="c", subcore_axis_name="s",
                                  num_cores=NC, num_subcores=NS)
    dot_run = functools.partial(
        pl.kernel,
        out_type=jax.ShapeDtypeStruct((B,), jnp.float32),
        mesh=mesh,
        compiler_params=pltpu.CompilerParams(needs_layout_passes=False),
        scratch_types=[
            pltpu.VMEM((BPW,), jnp.int32),
            pltpu.VMEM((BPW,), jnp.int32),
            pltpu.VMEM((NBUF, C, F), jnp.float32),
            pltpu.VMEM((NBUF, C, F), jnp.float32),
            pltpu.VMEM((L, MPAD), jnp.float32),
            pltpu.VMEM((BPW,), jnp.float32),
            pltpu.SemaphoreType.DMA((NBUF,)),
        ],
    )(_dot_body)
    bias_run = functools.partial(
        pl.kernel,
        out_type=jax.ShapeDtypeStruct((B,), jnp.float32),
        mesh=mesh,
        compiler_params=pltpu.CompilerParams(needs_layout_passes=False),
        scratch_types=[
            pltpu.VMEM((BPW,), jnp.int32),
            pltpu.VMEM((BPW,), jnp.int32),
            pltpu.VMEM((BPW,), jnp.float32),
            pltpu.VMEM((BPW,), jnp.float32),
            pltpu.VMEM((BPW,), jnp.float32),
            pltpu.SemaphoreType.DMA,
        ],
    )(_bias_body)
    dots = dot_run(user_id, item_id, P, Q)
    out = bias_run(user_id, item_id,
                   user_bias.reshape(-1), item_bias.reshape(-1), dots)
    return out.reshape(B, 1)
